# Initial kernel scaffold; baseline (speedup 1.0000x reference)
import jax
import jax.numpy as jnp
from jax.experimental import pallas as pl


def kernel(dense_x, discrete_x, dense_layer_W, dense_layer_b, discrete_layer_tables, dense_embedding_W, dense_embedding_b, discrete_embedding_tables, attn_W, attn_W_b, attn_h, pw_out_W, pw_out_b, out_W, out_b):
    raise NotImplementedError("write your pallas kernel here")


# speedup vs baseline: 1.5095x; 1.5095x over previous
import functools

import jax
import jax.numpy as jnp
import numpy as np
from jax import lax
from jax.experimental import pallas as pl
from jax.experimental.pallas import tpu as pltpu
from jax.experimental.pallas import tpu_sc as plsc

_B = 16384
_NS = 26
_ND = 13
_V = 100000
_D = 16
_NF = _NS + _ND
_PAIRS = _NF * (_NF - 1) // 2

_NC = 2
_NSUB = 16
_NW = _NC * _NSUB
_TOT = _B * _NS
_PER_W = _TOT // _NW
_CH = 1024
_CROWS = _CH // 128
_NCH = _PER_W // _CH

_BB = 256


def _sc_gather(emb_tab, lin_tab, idx2, lidx2):
    mesh = plsc.VectorSubcoreMesh(core_axis_name="c", subcore_axis_name="s")

    @functools.partial(
        pl.kernel,
        mesh=mesh,
        compiler_params=pltpu.CompilerParams(use_tc_tiling_on_sc=False),
        out_type=(
            jax.ShapeDtypeStruct((_TOT, _D), jnp.float32),
            jax.ShapeDtypeStruct((_TOT, _D), jnp.float32),
        ),
        scratch_types=[
            pltpu.VMEM((_CROWS, 128), jnp.int32),
            pltpu.VMEM((_CROWS, 128), jnp.int32),
            pltpu.VMEM((_CH, _D), jnp.float32),
            pltpu.VMEM((_CH, _D), jnp.float32),
            pltpu.SemaphoreType.DMA,
            pltpu.SemaphoreType.DMA,
        ],
    )
    def k(emb_hbm, lin_hbm, idx_hbm, lidx_hbm, emb_out, lrows_out,
          idx_v, lidx_v, rows_v, lrows_v, sem_e, sem_l):
        wid = lax.axis_index("s") * _NC + lax.axis_index("c")
        row_base = wid * (_PER_W // 128)
        base = wid * _PER_W

        def chunk(c, carry):
            row0 = row_base + c * _CROWS
            off = base + c * _CH
            pltpu.sync_copy(idx_hbm.at[pl.ds(row0, _CROWS)], idx_v)
            pltpu.sync_copy(lidx_hbm.at[pl.ds(row0, _CROWS)], lidx_v)
            cps = []
            for j in range(_CROWS):
                cps.append(pltpu.async_copy(
                    emb_hbm.at[idx_v.at[j]],
                    rows_v.at[pl.ds(j * 128, 128)], sem_e))
                cps.append(pltpu.async_copy(
                    lin_hbm.at[lidx_v.at[j]],
                    lrows_v.at[pl.ds(j * 128, 128)], sem_l))
            for cp in cps:
                cp.wait()
            pltpu.sync_copy(rows_v, emb_out.at[pl.ds(off, _CH)])
            pltpu.sync_copy(lrows_v, lrows_out.at[pl.ds(off, _CH)])
            return carry

        lax.fori_loop(0, _NCH, chunk, 0)

    return k(emb_tab, lin_tab, idx2, lidx2)


def _tc_body(dense_ref, emb_ref, lrows_ref, mod_ref, K26_ref, dpat_ref,
             dK_ref, dKb_ref, dlW_ref, dlb_ref,
             aW_ref, ab_ref, ahe_ref, R_ref, pwW_ref, pwb_ref, oW_ref, ob_ref,
             out_ref):
    dense = dense_ref[...]
    embf = emb_ref[...]
    dv = (jnp.dot(dense, dK_ref[...], preferred_element_type=jnp.float32)
          + dKb_ref[...])
    vf = jnp.concatenate([dv, embf], axis=1)

    R = R_ref[...]
    s = jnp.dot(vf, R, preferred_element_type=jnp.float32)
    q = jnp.dot(vf * vf, R, preferred_element_type=jnp.float32)
    ap = 0.5 * (s * s - q)

    t = jnp.dot(ap, aW_ref[...], preferred_element_type=jnp.float32)
    t = jnp.maximum(t + ab_ref[...], 0.0)
    ze = jnp.dot(t, ahe_ref[...], preferred_element_type=jnp.float32)
    ze = ze - jnp.max(ze, axis=-1, keepdims=True)
    ee = jnp.exp(ze)
    we = ee * (16.0 / jnp.sum(ee, axis=-1, keepdims=True))

    bb = dense.shape[0]
    acc = jnp.zeros((bb, _D), dtype=jnp.float32)
    off = 0
    for i in range(_NF - 1):
        m = _NF - 1 - i
        wslice = we[:, off * 16:(off + m) * 16]
        ta = jnp.concatenate(
            [jnp.zeros((bb, (i + 1) * _D), dtype=jnp.float32), wslice], axis=1)
        inner = jnp.dot(ta * vf, R, preferred_element_type=jnp.float32)
        acc = acc + vf[:, i * _D:(i + 1) * _D] * inner
        off += m

    pair_logit = (jnp.sum(acc * pwW_ref[...], axis=1, keepdims=True)
                  + pwb_ref[...])
    modexp = jnp.dot(mod_ref[...], K26_ref[...],
                     preferred_element_type=jnp.float32)
    onehot = jnp.where(modexp == dpat_ref[...], 1.0, 0.0)
    lin_sum = jnp.sum(lrows_ref[...] * onehot, axis=1, keepdims=True)
    lin_logit = (jnp.sum(dense * dlW_ref[...], axis=1, keepdims=True)
                 + dlb_ref[...] + lin_sum)
    oW = oW_ref[...]
    out_ref[...] = lin_logit * oW[:, 0:1] + pair_logit * oW[:, 1:2] + ob_ref[...]


def _tc_compute(dense_x, embf, lrowsf, mod2, K26, dpat, dK, dKb, dlW, dlb,
                aW, ab, ahe, R, pwW, pwb, oW, ob):
    grid = (_B // _BB,)
    full = lambda shape: pl.BlockSpec(shape, lambda g: tuple(0 for _ in shape))
    return pl.pallas_call(
        _tc_body,
        grid=grid,
        in_specs=[
            pl.BlockSpec((_BB, _ND), lambda g: (g, 0)),
            pl.BlockSpec((_BB, _NS * _D), lambda g: (g, 0)),
            pl.BlockSpec((_BB, _NS * _D), lambda g: (g, 0)),
            pl.BlockSpec((_BB, _NS), lambda g: (g, 0)),
            full((_NS, _NS * _D)),
            full((1, _NS * _D)),
            full((_ND, _ND * _D)),
            full((1, _ND * _D)),
            full((1, _ND)),
            full((1, 1)),
            full((_D, _D)),
            full((1, _D)),
            full((_D, _PAIRS * _D)),
            full((_NF * _D, _D)),
            full((1, _D)),
            full((1, 1)),
            full((1, 2)),
            full((1, 1)),
        ],
        out_specs=pl.BlockSpec((_BB, 1), lambda g: (g, 0)),
        out_shape=jax.ShapeDtypeStruct((_B, 1), jnp.float32),
    )(dense_x, embf, lrowsf, mod2, K26, dpat, dK, dKb, dlW, dlb,
      aW, ab, ahe, R, pwW, pwb, oW, ob)


def kernel(dense_x, discrete_x, dense_layer_W, dense_layer_b,
           discrete_layer_tables, dense_embedding_W, dense_embedding_b,
           discrete_embedding_tables, attn_W, attn_W_b, attn_h,
           pw_out_W, pw_out_b, out_W, out_b):
    offs = (jnp.arange(_NS, dtype=jnp.int32) * _V)[None, :]
    idx = discrete_x.astype(jnp.int32) + offs
    idx2 = idx.reshape(_TOT // 128, 128)
    lidx2 = (idx // 16).reshape(_TOT // 128, 128)
    mod2 = (idx % 16).astype(jnp.float32)

    emb_tab = discrete_embedding_tables.reshape(_NS * _V, _D)
    lin_tab = discrete_layer_tables.reshape(_NS * _V // 16, 16)

    emb_flat, lrows_flat = _sc_gather(emb_tab, lin_tab, idx2, lidx2)
    embf = emb_flat.reshape(_B, _NS * _D)
    lrowsf = lrows_flat.reshape(_B, _NS * _D)

    eye13 = jnp.eye(_ND, dtype=jnp.float32)
    dK = (eye13[:, :, None] * dense_embedding_W.reshape(1, 1, _D)
          ).reshape(_ND, _ND * _D)
    dKb = jnp.tile(dense_embedding_b.reshape(1, _D), (1, _ND))
    ahe = jnp.repeat(attn_h, _D, axis=1)
    R = jnp.tile(jnp.eye(_D, dtype=jnp.float32), (_NF, 1))
    eye26 = jnp.eye(_NS, dtype=jnp.float32)
    K26 = (eye26[:, :, None] * jnp.ones((1, 1, _D), jnp.float32)
           ).reshape(_NS, _NS * _D)
    dpat = jnp.tile(jnp.arange(_D, dtype=jnp.float32), (_NS,))[None, :]

    return _tc_compute(
        dense_x, embf, lrowsf, mod2, K26, dpat, dK, dKb,
        dense_layer_W.reshape(1, _ND),
        dense_layer_b.reshape(1, 1),
        attn_W,
        attn_W_b.reshape(1, _D),
        ahe,
        R,
        pw_out_W.reshape(1, _D),
        pw_out_b.reshape(1, 1),
        out_W.reshape(1, 2),
        out_b.reshape(1, 1),
    )


# speedup vs baseline: 1.5919x; 1.0546x over previous
import functools

import jax
import jax.numpy as jnp
import numpy as np
from jax import lax
from jax.experimental import pallas as pl
from jax.experimental.pallas import tpu as pltpu
from jax.experimental.pallas import tpu_sc as plsc

_B = 16384
_NS = 26
_ND = 13
_V = 100000
_D = 16
_NF = _NS + _ND
_PAIRS = _NF * (_NF - 1) // 2

_NC = 2
_NSUB = 16
_NW = _NC * _NSUB
_TOT = _B * _NS
_PER_W = _TOT // _NW
_CH = 1024
_CROWS = _CH // 128
_NCH = _PER_W // _CH

_BB = 512


def _sc_gather(emb_tab, lin_tab, idx2, lidx2):
    mesh = plsc.VectorSubcoreMesh(core_axis_name="c", subcore_axis_name="s")

    @functools.partial(
        pl.kernel,
        mesh=mesh,
        compiler_params=pltpu.CompilerParams(use_tc_tiling_on_sc=False),
        out_type=(
            jax.ShapeDtypeStruct((_TOT, _D), jnp.float32),
            jax.ShapeDtypeStruct((_TOT, _D), jnp.float32),
        ),
        scratch_types=[
            pltpu.VMEM((_CROWS, 128), jnp.int32),
            pltpu.VMEM((_CROWS, 128), jnp.int32),
            pltpu.VMEM((_CH, _D), jnp.float32),
            pltpu.VMEM((_CH, _D), jnp.float32),
            pltpu.SemaphoreType.DMA,
            pltpu.SemaphoreType.DMA,
        ],
    )
    def k(emb_hbm, lin_hbm, idx_hbm, lidx_hbm, emb_out, lrows_out,
          idx_v, lidx_v, rows_v, lrows_v, sem_e, sem_l):
        wid = lax.axis_index("s") * _NC + lax.axis_index("c")
        row_base = wid * (_PER_W // 128)
        base = wid * _PER_W

        def chunk(c, carry):
            row0 = row_base + c * _CROWS
            off = base + c * _CH
            pltpu.sync_copy(idx_hbm.at[pl.ds(row0, _CROWS)], idx_v)
            pltpu.sync_copy(lidx_hbm.at[pl.ds(row0, _CROWS)], lidx_v)
            cps = []
            for j in range(_CROWS):
                cps.append(pltpu.async_copy(
                    emb_hbm.at[idx_v.at[j]],
                    rows_v.at[pl.ds(j * 128, 128)], sem_e))
                cps.append(pltpu.async_copy(
                    lin_hbm.at[lidx_v.at[j]],
                    lrows_v.at[pl.ds(j * 128, 128)], sem_l))
            for cp in cps:
                cp.wait()
            pltpu.sync_copy(rows_v, emb_out.at[pl.ds(off, _CH)])
            pltpu.sync_copy(lrows_v, lrows_out.at[pl.ds(off, _CH)])
            return carry

        lax.fori_loop(0, _NCH, chunk, 0)

    return k(emb_tab, lin_tab, idx2, lidx2)


def _tc_body(dense_ref, emb_ref, lrows_ref, mod_ref, K26_ref, dpat_ref,
             dK_ref, dKb_ref, dlW_ref, dlb_ref,
             aW_ref, ab_ref, ahe_ref, R_ref, pwW_ref, pwb_ref, oW_ref, ob_ref,
             out_ref):
    dense = dense_ref[...]
    embf = emb_ref[...]
    dv = (jnp.dot(dense, dK_ref[...], preferred_element_type=jnp.float32)
          + dKb_ref[...])
    vf = jnp.concatenate([dv, embf], axis=1)

    R = R_ref[...]
    s = jnp.dot(vf, R, preferred_element_type=jnp.float32)
    q = jnp.dot(vf * vf, R, preferred_element_type=jnp.float32)
    ap = 0.5 * (s * s - q)

    t = jnp.dot(ap, aW_ref[...], preferred_element_type=jnp.float32)
    t = jnp.maximum(t + ab_ref[...], 0.0)
    ze = jnp.dot(t, ahe_ref[...], preferred_element_type=jnp.float32)
    ze = ze - jnp.max(ze, axis=-1, keepdims=True)
    ee = jnp.exp(ze)
    we = ee * (16.0 / jnp.sum(ee, axis=-1, keepdims=True))

    bb = dense.shape[0]
    acc = jnp.zeros((bb, _D), dtype=jnp.float32)
    off = 0
    for i in range(_NF - 1):
        m = _NF - 1 - i
        wslice = we[:, off * 16:(off + m) * 16]
        ta = jnp.concatenate(
            [jnp.zeros((bb, (i + 1) * _D), dtype=jnp.float32), wslice], axis=1)
        inner = jnp.dot(ta * vf, R, preferred_element_type=jnp.float32)
        acc = acc + vf[:, i * _D:(i + 1) * _D] * inner
        off += m

    pair_logit = (jnp.sum(acc * pwW_ref[...], axis=1, keepdims=True)
                  + pwb_ref[...])
    modexp = jnp.dot(mod_ref[...], K26_ref[...],
                     preferred_element_type=jnp.float32)
    onehot = jnp.where(modexp == dpat_ref[...], 1.0, 0.0)
    lin_sum = jnp.sum(lrows_ref[...] * onehot, axis=1, keepdims=True)
    lin_logit = (jnp.sum(dense * dlW_ref[...], axis=1, keepdims=True)
                 + dlb_ref[...] + lin_sum)
    oW = oW_ref[...]
    out_ref[...] = lin_logit * oW[:, 0:1] + pair_logit * oW[:, 1:2] + ob_ref[...]


def _tc_compute(dense_x, embf, lrowsf, mod2, K26, dpat, dK, dKb, dlW, dlb,
                aW, ab, ahe, R, pwW, pwb, oW, ob):
    grid = (_B // _BB,)
    full = lambda shape: pl.BlockSpec(shape, lambda g: tuple(0 for _ in shape))
    return pl.pallas_call(
        _tc_body,
        grid=grid,
        in_specs=[
            pl.BlockSpec((_BB, _ND), lambda g: (g, 0)),
            pl.BlockSpec((_BB, _NS * _D), lambda g: (g, 0)),
            pl.BlockSpec((_BB, _NS * _D), lambda g: (g, 0)),
            pl.BlockSpec((_BB, _NS), lambda g: (g, 0)),
            full((_NS, _NS * _D)),
            full((1, _NS * _D)),
            full((_ND, _ND * _D)),
            full((1, _ND * _D)),
            full((1, _ND)),
            full((1, 1)),
            full((_D, _D)),
            full((1, _D)),
            full((_D, _PAIRS * _D)),
            full((_NF * _D, _D)),
            full((1, _D)),
            full((1, 1)),
            full((1, 2)),
            full((1, 1)),
        ],
        out_specs=pl.BlockSpec((_BB, 1), lambda g: (g, 0)),
        out_shape=jax.ShapeDtypeStruct((_B, 1), jnp.float32),
    )(dense_x, embf, lrowsf, mod2, K26, dpat, dK, dKb, dlW, dlb,
      aW, ab, ahe, R, pwW, pwb, oW, ob)


def kernel(dense_x, discrete_x, dense_layer_W, dense_layer_b,
           discrete_layer_tables, dense_embedding_W, dense_embedding_b,
           discrete_embedding_tables, attn_W, attn_W_b, attn_h,
           pw_out_W, pw_out_b, out_W, out_b):
    offs = (jnp.arange(_NS, dtype=jnp.int32) * _V)[None, :]
    idx = discrete_x.astype(jnp.int32) + offs
    idx2 = idx.reshape(_TOT // 128, 128)
    lidx2 = (idx // 16).reshape(_TOT // 128, 128)
    mod2 = (idx % 16).astype(jnp.float32)

    emb_tab = discrete_embedding_tables.reshape(_NS * _V, _D)
    lin_tab = discrete_layer_tables.reshape(_NS * _V // 16, 16)

    emb_flat, lrows_flat = _sc_gather(emb_tab, lin_tab, idx2, lidx2)
    embf = emb_flat.reshape(_B, _NS * _D)
    lrowsf = lrows_flat.reshape(_B, _NS * _D)

    eye13 = jnp.eye(_ND, dtype=jnp.float32)
    dK = (eye13[:, :, None] * dense_embedding_W.reshape(1, 1, _D)
          ).reshape(_ND, _ND * _D)
    dKb = jnp.tile(dense_embedding_b.reshape(1, _D), (1, _ND))
    ahe = jnp.repeat(attn_h, _D, axis=1)
    R = jnp.tile(jnp.eye(_D, dtype=jnp.float32), (_NF, 1))
    eye26 = jnp.eye(_NS, dtype=jnp.float32)
    K26 = (eye26[:, :, None] * jnp.ones((1, 1, _D), jnp.float32)
           ).reshape(_NS, _NS * _D)
    dpat = jnp.tile(jnp.arange(_D, dtype=jnp.float32), (_NS,))[None, :]

    return _tc_compute(
        dense_x, embf, lrowsf, mod2, K26, dpat, dK, dKb,
        dense_layer_W.reshape(1, _ND),
        dense_layer_b.reshape(1, 1),
        attn_W,
        attn_W_b.reshape(1, _D),
        ahe,
        R,
        pw_out_W.reshape(1, _D),
        pw_out_b.reshape(1, 1),
        out_W.reshape(1, 2),
        out_b.reshape(1, 1),
    )
